# baseline (speedup 1.0000x reference)
import functools

import jax
import jax.numpy as jnp
from jax import lax
from jax.experimental import pallas as pl
from jax.experimental.pallas import tpu as pltpu
from jax.experimental.pallas import tpu_sc as plsc

NC = 2
NS = 16
NW = NC * NS
LANES = 16
BATCH = 16384
EMBED = 64
BPW = BATCH // NW
CHUNK = 128
NCH = BPW // CHUNK


def _sc_gather_partial(uidx, iidx, user_embedding, user_bias_flat,
                       item_embedding, item_bias_flat):
    mesh = plsc.VectorSubcoreMesh(
        core_axis_name="c", subcore_axis_name="s",
        num_cores=NC, num_subcores=NS)

    @functools.partial(
        pl.kernel,
        out_type=(
            jax.ShapeDtypeStruct((NW, LANES), jnp.float32),
            jax.ShapeDtypeStruct((NW, NCH, CHUNK), jnp.float32),
            jax.ShapeDtypeStruct((NW, NCH, CHUNK), jnp.float32),
        ),
        mesh=mesh,
        compiler_params=pltpu.CompilerParams(use_tc_tiling_on_sc=False),
        scratch_types=[
            pltpu.VMEM((NCH, CHUNK), jnp.int32),
            pltpu.VMEM((NCH, CHUNK), jnp.int32),
            pltpu.VMEM((BPW, EMBED), jnp.float32),
            pltpu.VMEM((BPW, EMBED), jnp.float32),
            pltpu.VMEM((NCH, CHUNK), jnp.float32),
            pltpu.VMEM((NCH, CHUNK), jnp.float32),
            pltpu.VMEM((LANES,), jnp.float32),
            pltpu.SemaphoreType.DMA,
            pltpu.SemaphoreType.DMA,
            pltpu.SemaphoreType.DMA,
        ],
    )
    def sc_kernel(uidx_h, iidx_h, uemb_h, ubias_h, iemb_h, ibias_h,
                  parts_h, ubg_h, ibg_h,
                  idxu_v, idxi_v, urows_v, vrows_v, ub_v, ib_v, acc_v,
                  sem_u, sem_v, sem_b):
        wid = lax.axis_index("s") * NC + lax.axis_index("c")
        pltpu.sync_copy(uidx_h.at[wid], idxu_v)
        pltpu.sync_copy(iidx_h.at[wid], idxi_v)
        copies = []
        for j in range(NCH):
            copies.append(pltpu.async_copy(
                uemb_h.at[idxu_v.at[j]], urows_v.at[pl.ds(j * CHUNK, CHUNK)],
                sem_u))
            copies.append(pltpu.async_copy(
                iemb_h.at[idxi_v.at[j]], vrows_v.at[pl.ds(j * CHUNK, CHUNK)],
                sem_v))
            copies.append(pltpu.async_copy(
                ubias_h.at[idxu_v.at[j]], ub_v.at[j], sem_b))
            copies.append(pltpu.async_copy(
                ibias_h.at[idxi_v.at[j]], ib_v.at[j], sem_b))
        for c in copies:
            c.wait()
        pltpu.sync_copy(ub_v, ubg_h.at[wid])
        pltpu.sync_copy(ib_v, ibg_h.at[wid])

        def body(b, acc):
            for j in range(EMBED // LANES):
                acc = acc + (urows_v[b, pl.ds(j * LANES, LANES)]
                             * vrows_v[b, pl.ds(j * LANES, LANES)])
            return acc

        acc = lax.fori_loop(0, BPW, body, jnp.zeros((LANES,), jnp.float32))
        acc_v[...] = acc
        pltpu.sync_copy(acc_v, parts_h.at[wid])

    return sc_kernel(uidx, iidx, user_embedding, user_bias_flat,
                     item_embedding, item_bias_flat)


def _tc_finish(parts2d, ub2d, ib2d):
    def tc_body(parts_ref, ub_ref, ib_ref, out_ref):
        s = jnp.sum(parts_ref[...])
        x = ub_ref[...] + ib_ref[...] + s
        out_ref[...] = 1.0 / (1.0 + jnp.exp(-x))

    return pl.pallas_call(
        tc_body,
        out_shape=jax.ShapeDtypeStruct(ub2d.shape, jnp.float32),
    )(parts2d, ub2d, ib2d)


def kernel(inputs, user_embedding, user_bias, item_embedding, item_bias):
    uidx = inputs[:, 0].reshape(NW, NCH, CHUNK)
    iidx = inputs[:, 1].reshape(NW, NCH, CHUNK)
    parts, ubg, ibg = _sc_gather_partial(
        uidx, iidx, user_embedding, user_bias.reshape(-1),
        item_embedding, item_bias.reshape(-1))
    out2d = _tc_finish(parts.reshape(NW * LANES // 128, 128),
                       ubg.reshape(128, 128), ibg.reshape(128, 128))
    return out2d.reshape(BATCH, 1)
